# baseline (speedup 1.0000x reference)
import functools

import jax
import jax.numpy as jnp
from jax import lax
from jax.experimental import pallas as pl
from jax.experimental.pallas import tpu as pltpu
from jax.experimental.pallas import tpu_sc as plsc

_NC = 2
_NS = 16
_LANES = 16
_K = 64
_NB = 4
_PF = 2


def _padded_ept(e: int) -> int:
    ept = -(-e // (_NC * _NS))
    blk = _K * _NB
    return -(-ept // blk) * blk


def _row_slabs(n: int):
    row_step = ((n // _NS) // 8) * 8
    row_len = n - (_NS - 1) * row_step
    assert row_len % 8 == 0 and row_len >= row_step
    return row_step, row_len


def _make_sc_gather_scale(n: int, d: int, e: int):
    assert n % _NS == 0 and d % _LANES == 0 and _K % _LANES == 0
    pept = _padded_ept(e)
    nchunk = pept // _K
    row_step, row_len = _row_slabs(n)
    mesh = plsc.VectorSubcoreMesh(core_axis_name="c", subcore_axis_name="s")

    @functools.partial(
        pl.kernel,
        out_type=jax.ShapeDtypeStruct((_NC * _NS * pept, d), jnp.float32),
        mesh=mesh,
        scratch_types=[
            pltpu.VMEM((pept,), jnp.int32),
            pltpu.VMEM((_NB, _K), jnp.float32),
            pltpu.VMEM((_NB, _K, d), jnp.float32),
            pltpu.VMEM_SHARED((n, d), jnp.float32),
            [pltpu.SemaphoreType.DMA] * _NB,
            [pltpu.SemaphoreType.DMA] * _NB,
        ],
    )
    def sc_gather_scale(x_hbm, src_hbm, val_hbm, out_hbm,
                        src_v, val_v, rows_v, x_sh, gsem, vsem):
        c = lax.axis_index("c")
        s = lax.axis_index("s")
        wid = c * _NS + s
        row0 = s * row_step
        pltpu.sync_copy(x_hbm.at[pl.ds(row0, row_len)],
                        x_sh.at[pl.ds(row0, row_len)])
        pltpu.sync_copy(src_hbm.at[wid], src_v)
        plsc.subcore_barrier()

        base = wid * pept

        def fetch(j, q):
            pltpu.async_copy(x_sh.at[src_v.at[pl.ds(j * _K, _K)]],
                             rows_v.at[q], gsem[q])
            pltpu.async_copy(val_hbm.at[wid, j], val_v.at[q], vsem[q])

        def fetch_wait(b):
            pltpu.make_async_copy(x_hbm.at[pl.ds(0, _K)], rows_v.at[b],
                                  gsem[b]).wait()
            pltpu.make_async_copy(val_hbm.at[0, 0], val_v.at[b],
                                  vsem[b]).wait()

        for b in range(_PF):
            fetch(b, b)

        def block_body(jo, carry):
            for b in range(_NB):
                i = jo * _NB + b
                q = (b + _PF) % _NB

                @pl.when(i + _PF < nchunk)
                def _prefetch():
                    fetch(i + _PF, q)

                fetch_wait(b)
                rows_b = rows_v.at[b]

                def scale_body(g, carry2):
                    vv = val_v[b, pl.ds(g * _LANES, _LANES)]
                    for t in range(_LANES):
                        v = vv[t]
                        r = g * _LANES + t
                        for f in range(d // _LANES):
                            sl = pl.ds(f * _LANES, _LANES)
                            rows_b[r, sl] = rows_b[r, sl] * v
                    return carry2

                lax.fori_loop(0, _K // _LANES, scale_body, 0)
                pltpu.sync_copy(rows_b, out_hbm.at[pl.ds(base + i * _K, _K)])
            return carry

        lax.fori_loop(0, nchunk // _NB, block_body, 0)

    return sc_gather_scale


def _make_sc_scatter(n: int, d: int, e: int):
    pept = _padded_ept(e)
    nchunk = pept // _K
    row_step, row_len = _row_slabs(n)
    mesh = plsc.VectorSubcoreMesh(core_axis_name="c", subcore_axis_name="s")

    @functools.partial(
        pl.kernel,
        out_type=jax.ShapeDtypeStruct((_NC, n, d), jnp.float32),
        mesh=mesh,
        scratch_types=[
            pltpu.VMEM((_NB, _K), jnp.int32),
            pltpu.VMEM((_NB, _K, d), jnp.float32),
            pltpu.VMEM_SHARED((n, d), jnp.float32),
            [pltpu.SemaphoreType.DMA] * _NB,
            [pltpu.SemaphoreType.DMA] * _NB,
        ],
    )
    def sc_scatter(scaled_hbm, dst_hbm, zeros_hbm, out_hbm,
                   dst_v, rows_v, agg_sh, gsem, ssem):
        c = lax.axis_index("c")
        s = lax.axis_index("s")
        wid = c * _NS + s
        row0 = s * row_step
        pltpu.sync_copy(zeros_hbm.at[pl.ds(row0, row_len)],
                        agg_sh.at[pl.ds(row0, row_len)])
        plsc.subcore_barrier()

        base = wid * pept

        def fetch(j, q):
            pltpu.async_copy(scaled_hbm.at[pl.ds(base + j * _K, _K)],
                             rows_v.at[q], gsem[q])
            pltpu.async_copy(dst_hbm.at[wid, j], dst_v.at[q], gsem[q])

        def fetch_wait(b):
            pltpu.make_async_copy(scaled_hbm.at[pl.ds(0, _K)], rows_v.at[b],
                                  gsem[b]).wait()
            pltpu.make_async_copy(dst_hbm.at[0, 0], dst_v.at[b],
                                  gsem[b]).wait()

        def scatter_wait(q):
            pltpu.make_async_copy(rows_v.at[q], agg_sh.at[pl.ds(0, _K)],
                                  ssem[q]).wait()

        for b in range(_PF):
            fetch(b, b)

        def block_body(jo, carry):
            for b in range(_NB):
                i = jo * _NB + b
                q = (b + _PF) % _NB

                @pl.when(i + _PF < nchunk)
                def _prefetch():
                    @pl.when(i >= _NB - _PF)
                    def _drain():
                        scatter_wait(q)
                    fetch(i + _PF, q)

                fetch_wait(b)
                pltpu.async_copy(rows_v.at[b], agg_sh.at[dst_v.at[b]],
                                 ssem[b], add=True)
            return carry

        lax.fori_loop(0, nchunk // _NB, block_body, 0)
        for b in range(_NB):
            scatter_wait(b)
        plsc.subcore_barrier()
        pltpu.sync_copy(agg_sh.at[pl.ds(row0, row_len)],
                        out_hbm.at[c, pl.ds(row0, row_len)])

    return sc_scatter


def _dense_body(agg_ref, x_ref, wl_ref, wr_ref, bsum_ref, o_ref):
    a = agg_ref[0] + agg_ref[1]
    h = lax.dot_general(a, wl_ref[...], (((1,), (1,)), ((), ())),
                        preferred_element_type=jnp.float32)
    h = h + lax.dot_general(x_ref[...], wr_ref[...], (((1,), (1,)), ((), ())),
                            preferred_element_type=jnp.float32)
    h = h + bsum_ref[...]
    denom = jnp.maximum(jnp.sum(jnp.abs(h), axis=1, keepdims=True), 1e-12)
    o_ref[...] = h / denom


def _make_dense(n: int, d: int):
    blk = 400
    while n % blk or blk % 8:
        blk //= 2
    grid = n // blk
    return pl.pallas_call(
        _dense_body,
        grid=(grid,),
        in_specs=[
            pl.BlockSpec((_NC, blk, d), lambda i: (0, i, 0)),
            pl.BlockSpec((blk, d), lambda i: (i, 0)),
            pl.BlockSpec((d, d), lambda i: (0, 0)),
            pl.BlockSpec((d, d), lambda i: (0, 0)),
            pl.BlockSpec((1, d), lambda i: (0, 0)),
        ],
        out_specs=pl.BlockSpec((blk, d), lambda i: (i, 0)),
        out_shape=jax.ShapeDtypeStruct((n, d), jnp.float32),
    )


def kernel(x, edge_vals, W_l, b_l, W_r, b_r, edge_index):
    n, d = x.shape
    e = edge_vals.shape[0]
    nw = _NC * _NS
    pept = _padded_ept(e)
    nchunk = pept // _K
    pad = nw * pept - e

    def slab(a, fill):
        return jnp.pad(a, (0, pad), constant_values=fill).reshape(nw, pept)

    dst = slab(edge_index[0], 0).reshape(nw, nchunk, _K)
    src = slab(edge_index[1], 0)
    vals = slab(edge_vals, 0.0).reshape(nw, nchunk, _K)
    zeros = jnp.zeros((n, d), jnp.float32)
    scaled = _make_sc_gather_scale(n, d, e)(x, src, vals)
    partials = _make_sc_scatter(n, d, e)(scaled, dst, zeros)
    bsum = (b_l + b_r)[None, :]
    return _make_dense(n, d)(partials, x, W_l, W_r, bsum)


# speedup vs baseline: 1.1056x; 1.1056x over previous
import functools

import jax
import jax.numpy as jnp
from jax import lax
from jax.experimental import pallas as pl
from jax.experimental.pallas import tpu as pltpu
from jax.experimental.pallas import tpu_sc as plsc

_NC = 2
_NS = 16
_LANES = 16
_K = 64
_NB = 4
_PF = 2


def _padded_ept(e: int) -> int:
    ept = -(-e // (_NC * _NS))
    blk = _K * _NB
    return -(-ept // blk) * blk


def _row_slabs(n: int):
    row_step = ((n // _NS) // 8) * 8
    row_len = n - (_NS - 1) * row_step
    assert row_len % 8 == 0 and row_len >= row_step
    return row_step, row_len


def _make_sc_gather_scale(n: int, d: int, e: int):
    assert n % _NS == 0 and d % _LANES == 0 and _K % _LANES == 0
    pept = _padded_ept(e)
    nchunk = pept // _K
    row_step, row_len = _row_slabs(n)
    mesh = plsc.VectorSubcoreMesh(core_axis_name="c", subcore_axis_name="s")

    @functools.partial(
        pl.kernel,
        out_type=jax.ShapeDtypeStruct((_NC * _NS * pept, d), jnp.float32),
        mesh=mesh,
        scratch_types=[
            pltpu.VMEM((pept,), jnp.int32),
            pltpu.VMEM((_NB, _K), jnp.float32),
            pltpu.VMEM((_NB, _K, d), jnp.float32),
            pltpu.VMEM_SHARED((n, d), jnp.float32),
            [pltpu.SemaphoreType.DMA] * _NB,
            [pltpu.SemaphoreType.DMA] * _NB,
            [pltpu.SemaphoreType.DMA] * _NB,
        ],
    )
    def sc_gather_scale(x_hbm, src_hbm, val_hbm, out_hbm,
                        src_v, val_v, rows_v, x_sh, gsem, vsem, ssem):
        c = lax.axis_index("c")
        s = lax.axis_index("s")
        wid = c * _NS + s
        row0 = s * row_step
        pltpu.sync_copy(x_hbm.at[pl.ds(row0, row_len)],
                        x_sh.at[pl.ds(row0, row_len)])
        pltpu.sync_copy(src_hbm.at[wid], src_v)
        plsc.subcore_barrier()

        base = wid * pept

        def fetch(j, q):
            pltpu.async_copy(x_sh.at[src_v.at[pl.ds(j * _K, _K)]],
                             rows_v.at[q], gsem[q])
            pltpu.async_copy(val_hbm.at[wid, j], val_v.at[q], vsem[q])

        def fetch_wait(b):
            pltpu.make_async_copy(x_hbm.at[pl.ds(0, _K)], rows_v.at[b],
                                  gsem[b]).wait()
            pltpu.make_async_copy(val_hbm.at[0, 0], val_v.at[b],
                                  vsem[b]).wait()

        def store_wait(q):
            pltpu.make_async_copy(rows_v.at[q], out_hbm.at[pl.ds(0, _K)],
                                  ssem[q]).wait()

        for b in range(_PF):
            fetch(b, b)

        def block_body(jo, carry):
            for b in range(_NB):
                i = jo * _NB + b
                q = (b + _PF) % _NB

                @pl.when(i + _PF < nchunk)
                def _prefetch():
                    @pl.when(i >= _NB - _PF)
                    def _drain():
                        store_wait(q)
                    fetch(i + _PF, q)

                fetch_wait(b)
                rows_b = rows_v.at[b]

                def scale_body(g, carry2):
                    vv = val_v[b, pl.ds(g * _LANES, _LANES)]
                    for t in range(_LANES):
                        v = vv[t]
                        r = g * _LANES + t
                        for f in range(d // _LANES):
                            sl = pl.ds(f * _LANES, _LANES)
                            rows_b[r, sl] = rows_b[r, sl] * v
                    return carry2

                lax.fori_loop(0, _K // _LANES, scale_body, 0)
                pltpu.async_copy(rows_b, out_hbm.at[pl.ds(base + i * _K, _K)],
                                 ssem[b])
            return carry

        lax.fori_loop(0, nchunk // _NB, block_body, 0)
        for b in range(_NB):
            store_wait(b)

    return sc_gather_scale


def _make_sc_scatter(n: int, d: int, e: int):
    pept = _padded_ept(e)
    nchunk = pept // _K
    row_step, row_len = _row_slabs(n)
    mesh = plsc.VectorSubcoreMesh(core_axis_name="c", subcore_axis_name="s")

    @functools.partial(
        pl.kernel,
        out_type=jax.ShapeDtypeStruct((_NC, n, d), jnp.float32),
        mesh=mesh,
        scratch_types=[
            pltpu.VMEM((_NB, _K), jnp.int32),
            pltpu.VMEM((_NB, _K, d), jnp.float32),
            pltpu.VMEM_SHARED((n, d), jnp.float32),
            [pltpu.SemaphoreType.DMA] * _NB,
            [pltpu.SemaphoreType.DMA] * _NB,
        ],
    )
    def sc_scatter(scaled_hbm, dst_hbm, zeros_hbm, out_hbm,
                   dst_v, rows_v, agg_sh, gsem, ssem):
        c = lax.axis_index("c")
        s = lax.axis_index("s")
        wid = c * _NS + s
        row0 = s * row_step
        pltpu.sync_copy(zeros_hbm.at[pl.ds(row0, row_len)],
                        agg_sh.at[pl.ds(row0, row_len)])
        plsc.subcore_barrier()

        base = wid * pept

        def fetch(j, q):
            pltpu.async_copy(scaled_hbm.at[pl.ds(base + j * _K, _K)],
                             rows_v.at[q], gsem[q])
            pltpu.async_copy(dst_hbm.at[wid, j], dst_v.at[q], gsem[q])

        def fetch_wait(b):
            pltpu.make_async_copy(scaled_hbm.at[pl.ds(0, _K)], rows_v.at[b],
                                  gsem[b]).wait()
            pltpu.make_async_copy(dst_hbm.at[0, 0], dst_v.at[b],
                                  gsem[b]).wait()

        def scatter_wait(q):
            pltpu.make_async_copy(rows_v.at[q], agg_sh.at[pl.ds(0, _K)],
                                  ssem[q]).wait()

        for b in range(_PF):
            fetch(b, b)

        def block_body(jo, carry):
            for b in range(_NB):
                i = jo * _NB + b
                q = (b + _PF) % _NB

                @pl.when(i + _PF < nchunk)
                def _prefetch():
                    @pl.when(i >= _NB - _PF)
                    def _drain():
                        scatter_wait(q)
                    fetch(i + _PF, q)

                fetch_wait(b)
                pltpu.async_copy(rows_v.at[b], agg_sh.at[dst_v.at[b]],
                                 ssem[b], add=True)
            return carry

        lax.fori_loop(0, nchunk // _NB, block_body, 0)
        for b in range(_NB):
            scatter_wait(b)
        plsc.subcore_barrier()
        pltpu.sync_copy(agg_sh.at[pl.ds(row0, row_len)],
                        out_hbm.at[c, pl.ds(row0, row_len)])

    return sc_scatter


def _dense_body(agg_ref, x_ref, wl_ref, wr_ref, bsum_ref, o_ref):
    a = agg_ref[0] + agg_ref[1]
    h = lax.dot_general(a, wl_ref[...], (((1,), (1,)), ((), ())),
                        preferred_element_type=jnp.float32)
    h = h + lax.dot_general(x_ref[...], wr_ref[...], (((1,), (1,)), ((), ())),
                            preferred_element_type=jnp.float32)
    h = h + bsum_ref[...]
    denom = jnp.maximum(jnp.sum(jnp.abs(h), axis=1, keepdims=True), 1e-12)
    o_ref[...] = h / denom


def _make_dense(n: int, d: int):
    blk = 400
    while n % blk or blk % 8:
        blk //= 2
    grid = n // blk
    return pl.pallas_call(
        _dense_body,
        grid=(grid,),
        in_specs=[
            pl.BlockSpec((_NC, blk, d), lambda i: (0, i, 0)),
            pl.BlockSpec((blk, d), lambda i: (i, 0)),
            pl.BlockSpec((d, d), lambda i: (0, 0)),
            pl.BlockSpec((d, d), lambda i: (0, 0)),
            pl.BlockSpec((1, d), lambda i: (0, 0)),
        ],
        out_specs=pl.BlockSpec((blk, d), lambda i: (i, 0)),
        out_shape=jax.ShapeDtypeStruct((n, d), jnp.float32),
    )


def kernel(x, edge_vals, W_l, b_l, W_r, b_r, edge_index):
    n, d = x.shape
    e = edge_vals.shape[0]
    nw = _NC * _NS
    pept = _padded_ept(e)
    nchunk = pept // _K
    pad = nw * pept - e

    def slab(a, fill):
        return jnp.pad(a, (0, pad), constant_values=fill).reshape(nw, pept)

    dst = slab(edge_index[0], 0).reshape(nw, nchunk, _K)
    src = slab(edge_index[1], 0)
    vals = slab(edge_vals, 0.0).reshape(nw, nchunk, _K)
    zeros = jnp.zeros((n, d), jnp.float32)
    scaled = _make_sc_gather_scale(n, d, e)(x, src, vals)
    partials = _make_sc_scatter(n, d, e)(scaled, dst, zeros)
    bsum = (b_l + b_r)[None, :]
    return _make_dense(n, d)(partials, x, W_l, W_r, bsum)
